# Initial kernel scaffold; baseline (speedup 1.0000x reference)
import jax
import jax.numpy as jnp
from jax.experimental import pallas as pl


def kernel(x, edge_index1, edge_weight1, edge_index2, edge_weight2, W_in1, b_in1, W_in2, b_in2, fW1, fW2, W_out1, b_out1, W_out2, b_out2):
    raise NotImplementedError("write your pallas kernel here")


# speedup vs baseline: 2.7104x; 2.7104x over previous
import functools

import jax
import jax.numpy as jnp
from jax import lax
from jax.experimental import pallas as pl
from jax.experimental.pallas import tpu as pltpu
from jax.experimental.pallas import tpu_sc as plsc

_N = 10000
_NP = 10240
_H = 64
_NTILES = 16
_ROWS_PER_TILE = _NP // _NTILES
_RCH = 128
_C = 128


def _input_linear(x, W1, b1, W2, b2):
    N, D = x.shape
    H = W1.shape[1]
    BN = 1000
    nb = N // BN
    Ws = jnp.stack([W1, W2])
    bs = jnp.stack([b1.reshape(1, H), b2.reshape(1, H)])

    def body(x_ref, w_ref, b_ref, o_ref):
        acc = jnp.dot(x_ref[...], w_ref[0],
                      preferred_element_type=jnp.float32)
        o_ref[...] = acc + b_ref[0]

    return pl.pallas_call(
        body,
        grid=(2, nb),
        in_specs=[
            pl.BlockSpec((BN, D), lambda i, b: (b, 0)),
            pl.BlockSpec((1, D, H), lambda i, b: (i, 0, 0)),
            pl.BlockSpec((1, 1, H), lambda i, b: (i, 0, 0)),
        ],
        out_specs=pl.BlockSpec((BN, H), lambda i, b: (i * nb + b, 0)),
        out_shape=jax.ShapeDtypeStruct((2 * N, H), jnp.float32),
    )(x, Ws, bs)


def _output_linear(hid, Wo1, bo1, Wo2, bo2):
    twoN, H = hid.shape
    N = twoN // 2
    OUT = Wo2.shape[1]
    BN = 1000
    nb = N // BN
    Wa = Wo1[:H]
    Wb = Wo1[H:]

    def body(h1_ref, h2_ref, wa_ref, wb_ref, b1_ref, w2_ref, b2_ref, o_ref):
        t = (jnp.dot(h1_ref[...], wa_ref[...], preferred_element_type=jnp.float32)
             + jnp.dot(h2_ref[...], wb_ref[...], preferred_element_type=jnp.float32)
             + b1_ref[...])
        t = jnp.where(t >= 0, t, 0.01 * t)
        o = jnp.dot(t, w2_ref[...], preferred_element_type=jnp.float32) + b2_ref[...]
        o_ref[...] = jnp.where(o >= 0, o, 0.01 * o)

    return pl.pallas_call(
        body,
        grid=(nb,),
        in_specs=[
            pl.BlockSpec((BN, H), lambda b: (b, 0)),
            pl.BlockSpec((BN, H), lambda b: (nb + b, 0)),
            pl.BlockSpec((H, H), lambda b: (0, 0)),
            pl.BlockSpec((H, H), lambda b: (0, 0)),
            pl.BlockSpec((1, H), lambda b: (0, 0)),
            pl.BlockSpec((H, OUT), lambda b: (0, 0)),
            pl.BlockSpec((1, OUT), lambda b: (0, 0)),
        ],
        out_specs=pl.BlockSpec((BN, OUT), lambda b: (b, 0)),
        out_shape=jax.ShapeDtypeStruct((N, OUT), jnp.float32),
    )(hid, hid, Wa, Wb, bo1.reshape(1, H), Wo2, bo2.reshape(1, OUT))


def _lane_splat(vec, lane):
    idx = jnp.full((16, 1), lane, jnp.int32)
    return lax.gather(
        vec, idx,
        lax.GatherDimensionNumbers(
            offset_dims=(), collapsed_slice_dims=(0,), start_index_map=(0,)),
        (1,), mode=lax.GatherScatterMode.PROMISE_IN_BOUNDS)


def _make_prop_kernel(K, n_chunks):
    mesh = plsc.VectorSubcoreMesh(core_axis_name="c", subcore_axis_name="s")

    @functools.partial(
        pl.kernel,
        out_type=jax.ShapeDtypeStruct((2 * _NP, _H), jnp.float32),
        mesh=mesh,
        scratch_types=[
            pltpu.HBM((2 * _NP, _H), jnp.float32),
            pltpu.VMEM_SHARED((_NP, _H), jnp.float32),
            pltpu.VMEM((n_chunks, _C), jnp.int32),
            pltpu.VMEM((n_chunks, _C), jnp.int32),
            pltpu.VMEM((n_chunks, _C), jnp.float32),
            pltpu.VMEM((_C, _H), jnp.float32),
            pltpu.VMEM((_C, _H), jnp.float32),
            pltpu.VMEM((_C, _H), jnp.float32),
            pltpu.VMEM((16, 16), jnp.float32),
            pltpu.SemaphoreType.DMA,
        ],
        compiler_params=pltpu.CompilerParams(use_tc_tiling_on_sc=False),
    )
    def prop(h_in, src_e, dst_e, w_e, fw, hid_out,
             h_cur, acc_sh, src_v, dst_v, w_v,
             rows_v, zero_v, buf_a, fw_v, sem):
        cid = lax.axis_index("c")
        sid = lax.axis_index("s")
        base = sid * _ROWS_PER_TILE
        gbase = cid * _NP + base

        pltpu.sync_copy(src_e.at[cid, sid], src_v)
        pltpu.sync_copy(dst_e.at[cid, sid], dst_v)
        pltpu.sync_copy(w_e.at[cid, sid], w_v)
        pltpu.sync_copy(fw.at[cid], fw_v)

        @pl.loop(0, _C)
        def _zfill(i):
            for f in range(_H // 16):
                zero_v[i, pl.ds(16 * f, 16)] = jnp.zeros((16,), jnp.float32)

        fw0 = fw_v[0]
        for c in range(_ROWS_PER_TILE // _RCH):
            r0 = base + c * _RCH
            g0 = gbase + c * _RCH
            pltpu.sync_copy(h_in.at[pl.ds(g0, _RCH)], rows_v.at[pl.ds(0, _RCH)])
            pltpu.sync_copy(rows_v.at[pl.ds(0, _RCH)], h_cur.at[pl.ds(g0, _RCH)])

            @pl.loop(0, _RCH)
            def _scale0(i):
                for f in range(_H // 16):
                    buf_a[i, pl.ds(16 * f, 16)] = rows_v[i, pl.ds(16 * f, 16)] * fw0

            pltpu.sync_copy(buf_a.at[pl.ds(0, _RCH)], hid_out.at[pl.ds(g0, _RCH)])
            pltpu.sync_copy(zero_v.at[pl.ds(0, _RCH)], acc_sh.at[pl.ds(r0, _RCH)])
        plsc.subcore_barrier()

        @pl.loop(0, K)
        def _round(k):
            @pl.loop(0, n_chunks)
            def _chunk(j):
                pltpu.async_copy(h_cur.at[src_v.at[j]], rows_v, sem).wait()

                @pl.loop(0, _C // 16)
                def _scale(g):
                    wv = w_v[j, pl.ds(g * 16, 16)]
                    for e in range(16):
                        w = _lane_splat(wv, e)
                        for f in range(_H // 16):
                            rows_v[g * 16 + e, pl.ds(16 * f, 16)] = (
                                rows_v[g * 16 + e, pl.ds(16 * f, 16)] * w)

                pltpu.sync_copy(rows_v, acc_sh.at[dst_v.at[j]], add=True)

            plsc.subcore_barrier()

            fwk = fw_v[k + 1]
            for c in range(_ROWS_PER_TILE // _RCH):
                r0 = base + c * _RCH
                g0 = gbase + c * _RCH
                pltpu.sync_copy(acc_sh.at[pl.ds(r0, _RCH)], buf_a.at[pl.ds(0, _RCH)])
                pltpu.sync_copy(buf_a.at[pl.ds(0, _RCH)], h_cur.at[pl.ds(g0, _RCH)])
                pltpu.sync_copy(hid_out.at[pl.ds(g0, _RCH)], rows_v.at[pl.ds(0, _RCH)])

                @pl.loop(0, _RCH)
                def _fma(i):
                    for f in range(_H // 16):
                        rows_v[i, pl.ds(16 * f, 16)] = (
                            rows_v[i, pl.ds(16 * f, 16)]
                            + buf_a[i, pl.ds(16 * f, 16)] * fwk)

                pltpu.sync_copy(rows_v.at[pl.ds(0, _RCH)], hid_out.at[pl.ds(g0, _RCH)])
                pltpu.sync_copy(zero_v.at[pl.ds(0, _RCH)], acc_sh.at[pl.ds(r0, _RCH)])
            plsc.subcore_barrier()

    return prop


def _prep_edges(edge_index, edge_weight, src_offset, n_chunks):
    E = edge_weight.shape[0]
    ep = _NTILES * n_chunks * _C
    pad = ep - E
    src = jnp.concatenate(
        [edge_index[1] + src_offset, jnp.zeros((pad,), jnp.int32)])
    dst = jnp.concatenate([edge_index[0], jnp.zeros((pad,), jnp.int32)])
    w = jnp.concatenate([edge_weight, jnp.zeros((pad,), jnp.float32)])
    return (src.reshape(_NTILES, n_chunks, _C),
            dst.reshape(_NTILES, n_chunks, _C),
            w.reshape(_NTILES, n_chunks, _C))


def kernel(x, edge_index1, edge_weight1, edge_index2, edge_weight2,
           W_in1, b_in1, W_in2, b_in2, fW1, fW2,
           W_out1, b_out1, W_out2, b_out2):
    N, D = x.shape
    E = edge_weight1.shape[0]
    K = fW1.shape[0] - 1
    assert N == _N and W_in1.shape[1] == _H

    n_chunks = -(-E // (_NTILES * _C))

    h2n = _input_linear(x, W_in1, b_in1, W_in2, b_in2)
    h_stack = jnp.concatenate([
        jnp.pad(h2n[:N], ((0, _NP - N), (0, 0))),
        jnp.pad(h2n[N:], ((0, _NP - N), (0, 0))),
    ])

    s1, d1, w1 = _prep_edges(edge_index1, edge_weight1, 0, n_chunks)
    s2, d2, w2 = _prep_edges(edge_index2, edge_weight2, _NP, n_chunks)
    src_e = jnp.stack([s1, s2])
    dst_e = jnp.stack([d1, d2])
    w_e = jnp.stack([w1, w2])
    fw = jnp.stack([
        jnp.zeros((16,), jnp.float32).at[:K + 1].set(fW1),
        jnp.zeros((16,), jnp.float32).at[:K + 1].set(fW2),
    ])
    fw = jnp.broadcast_to(fw[:, :, None], (2, 16, 16))

    prop = _make_prop_kernel(K, n_chunks)
    hid = prop(h_stack, src_e, dst_e, w_e, fw)
    hid = jnp.concatenate([hid[:N], hid[_NP:_NP + N]])

    return _output_linear(hid, W_out1, b_out1, W_out2, b_out2)


# speedup vs baseline: 3.3541x; 1.2375x over previous
import functools

import jax
import jax.numpy as jnp
from jax import lax
from jax.experimental import pallas as pl
from jax.experimental.pallas import tpu as pltpu
from jax.experimental.pallas import tpu_sc as plsc

_N = 10000
_NP = 10240
_H = 64
_NTILES = 16
_ROWS_PER_TILE = _NP // _NTILES
_RCH = 128
_C = 128


def _input_linear(x, W1, b1, W2, b2):
    N, D = x.shape
    H = W1.shape[1]
    BN = 1000
    nb = N // BN
    Ws = jnp.stack([W1, W2])
    bs = jnp.stack([b1.reshape(1, H), b2.reshape(1, H)])

    def body(x_ref, w_ref, b_ref, o_ref):
        acc = jnp.dot(x_ref[...], w_ref[0],
                      preferred_element_type=jnp.float32)
        o_ref[...] = acc + b_ref[0]

    return pl.pallas_call(
        body,
        grid=(2, nb),
        in_specs=[
            pl.BlockSpec((BN, D), lambda i, b: (b, 0)),
            pl.BlockSpec((1, D, H), lambda i, b: (i, 0, 0)),
            pl.BlockSpec((1, 1, H), lambda i, b: (i, 0, 0)),
        ],
        out_specs=pl.BlockSpec((BN, H), lambda i, b: (i * nb + b, 0)),
        out_shape=jax.ShapeDtypeStruct((2 * N, H), jnp.float32),
    )(x, Ws, bs)


def _output_linear(hid, Wo1, bo1, Wo2, bo2):
    twoN, H = hid.shape
    N = twoN // 2
    OUT = Wo2.shape[1]
    BN = 1000
    nb = N // BN
    Wa = Wo1[:H]
    Wb = Wo1[H:]

    def body(h1_ref, h2_ref, wa_ref, wb_ref, b1_ref, w2_ref, b2_ref, o_ref):
        t = (jnp.dot(h1_ref[...], wa_ref[...], preferred_element_type=jnp.float32)
             + jnp.dot(h2_ref[...], wb_ref[...], preferred_element_type=jnp.float32)
             + b1_ref[...])
        t = jnp.where(t >= 0, t, 0.01 * t)
        o = jnp.dot(t, w2_ref[...], preferred_element_type=jnp.float32) + b2_ref[...]
        o_ref[...] = jnp.where(o >= 0, o, 0.01 * o)

    return pl.pallas_call(
        body,
        grid=(nb,),
        in_specs=[
            pl.BlockSpec((BN, H), lambda b: (b, 0)),
            pl.BlockSpec((BN, H), lambda b: (nb + b, 0)),
            pl.BlockSpec((H, H), lambda b: (0, 0)),
            pl.BlockSpec((H, H), lambda b: (0, 0)),
            pl.BlockSpec((1, H), lambda b: (0, 0)),
            pl.BlockSpec((H, OUT), lambda b: (0, 0)),
            pl.BlockSpec((1, OUT), lambda b: (0, 0)),
        ],
        out_specs=pl.BlockSpec((BN, OUT), lambda b: (b, 0)),
        out_shape=jax.ShapeDtypeStruct((N, OUT), jnp.float32),
    )(hid, hid, Wa, Wb, bo1.reshape(1, H), Wo2, bo2.reshape(1, OUT))


def _lane_splat(vec, lane):
    idx = jnp.full((16, 1), lane, jnp.int32)
    return lax.gather(
        vec, idx,
        lax.GatherDimensionNumbers(
            offset_dims=(), collapsed_slice_dims=(0,), start_index_map=(0,)),
        (1,), mode=lax.GatherScatterMode.PROMISE_IN_BOUNDS)


def _make_prop_kernel(K, n_chunks):
    mesh = plsc.VectorSubcoreMesh(core_axis_name="c", subcore_axis_name="s")
    NB = 3

    @functools.partial(
        pl.kernel,
        out_type=jax.ShapeDtypeStruct((2 * _NP, _H), jnp.float32),
        mesh=mesh,
        scratch_types=[
            pltpu.HBM((2 * _NP, _H), jnp.float32),
            pltpu.VMEM_SHARED((_NP, _H), jnp.float32),
            pltpu.VMEM((n_chunks, _C), jnp.int32),
            pltpu.VMEM((n_chunks, _C), jnp.int32),
            pltpu.VMEM((n_chunks, _C), jnp.float32),
            pltpu.VMEM((NB, _C, _H), jnp.float32),
            pltpu.VMEM((16, 16), jnp.float32),
            pltpu.SemaphoreType.DMA,
            pltpu.SemaphoreType.DMA,
            pltpu.SemaphoreType.DMA,
            pltpu.SemaphoreType.DMA,
            pltpu.SemaphoreType.DMA,
            pltpu.SemaphoreType.DMA,
        ],
        compiler_params=pltpu.CompilerParams(use_tc_tiling_on_sc=False),
    )
    def prop(h_in, src_e, dst_e, w_e, fw, hid_out,
             h_cur, acc_sh, src_v, dst_v, w_v,
             rows_v, fw_v,
             gs0, gs1, gs2, ss0, ss1, ss2):
        gsems = [gs0, gs1, gs2]
        ssems = [ss0, ss1, ss2]
        cid = lax.axis_index("c")
        sid = lax.axis_index("s")
        base = sid * _ROWS_PER_TILE
        gbase = cid * _NP + base

        pltpu.sync_copy(src_e.at[cid, sid], src_v)
        pltpu.sync_copy(dst_e.at[cid, sid], dst_v)
        pltpu.sync_copy(w_e.at[cid, sid], w_v)
        pltpu.sync_copy(fw.at[cid], fw_v)

        @pl.loop(0, _C)
        def _zfill(i):
            for f in range(_H // 16):
                rows_v[2, i, pl.ds(16 * f, 16)] = jnp.zeros((16,), jnp.float32)

        fw0 = fw_v[0]
        for c in range(_ROWS_PER_TILE // _RCH):
            r0 = base + c * _RCH
            g0 = gbase + c * _RCH
            pltpu.sync_copy(h_in.at[pl.ds(g0, _RCH)], rows_v.at[0])
            pltpu.sync_copy(rows_v.at[0], h_cur.at[pl.ds(g0, _RCH)])

            @pl.loop(0, _RCH)
            def _scale0(i):
                for f in range(_H // 16):
                    rows_v[1, i, pl.ds(16 * f, 16)] = rows_v[0, i, pl.ds(16 * f, 16)] * fw0

            pltpu.sync_copy(rows_v.at[1], hid_out.at[pl.ds(g0, _RCH)])
            pltpu.sync_copy(rows_v.at[2], acc_sh.at[pl.ds(r0, _RCH)])
        plsc.subcore_barrier()

        @pl.loop(0, K)
        def _round(k):
            for b in range(NB):
                pltpu.async_copy(h_cur.at[src_v.at[b]], rows_v.at[b], gsems[b])

            @pl.loop(0, n_chunks, step=NB)
            def _grp(j0):
                for b in range(NB):
                    j = j0 + b
                    pltpu.make_async_copy(
                        h_cur.at[src_v.at[j]], rows_v.at[b], gsems[b]).wait()

                    @pl.loop(0, _C // 16)
                    def _scale(g):
                        wv = w_v[j, pl.ds(g * 16, 16)]
                        for e in range(16):
                            w = _lane_splat(wv, e)
                            for f in range(_H // 16):
                                rows_v[b, g * 16 + e, pl.ds(16 * f, 16)] = (
                                    rows_v[b, g * 16 + e, pl.ds(16 * f, 16)] * w)

                    pltpu.async_copy(
                        rows_v.at[b], acc_sh.at[dst_v.at[j]], ssems[b], add=True)
                    nxt = j + NB

                    @pl.when(nxt < n_chunks)
                    def _refill():
                        pltpu.make_async_copy(
                            rows_v.at[b], acc_sh.at[dst_v.at[j]], ssems[b]).wait()
                        pltpu.async_copy(
                            h_cur.at[src_v.at[nxt]], rows_v.at[b], gsems[b])

            for b in range(NB):
                pltpu.make_async_copy(
                    rows_v.at[b], acc_sh.at[dst_v.at[0]], ssems[b]).wait()
            plsc.subcore_barrier()

            @pl.loop(0, _C)
            def _zfill2(i):
                for f in range(_H // 16):
                    rows_v[2, i, pl.ds(16 * f, 16)] = jnp.zeros((16,), jnp.float32)

            fwk = fw_v[k + 1]
            for c in range(_ROWS_PER_TILE // _RCH):
                r0 = base + c * _RCH
                g0 = gbase + c * _RCH
                pltpu.sync_copy(acc_sh.at[pl.ds(r0, _RCH)], rows_v.at[1])
                pltpu.sync_copy(rows_v.at[1], h_cur.at[pl.ds(g0, _RCH)])
                pltpu.sync_copy(hid_out.at[pl.ds(g0, _RCH)], rows_v.at[0])

                @pl.loop(0, _RCH)
                def _fma(i):
                    for f in range(_H // 16):
                        rows_v[0, i, pl.ds(16 * f, 16)] = (
                            rows_v[0, i, pl.ds(16 * f, 16)]
                            + rows_v[1, i, pl.ds(16 * f, 16)] * fwk)

                pltpu.sync_copy(rows_v.at[0], hid_out.at[pl.ds(g0, _RCH)])
                pltpu.sync_copy(rows_v.at[2], acc_sh.at[pl.ds(r0, _RCH)])
            plsc.subcore_barrier()

    return prop


def _prep_edges(edge_index, edge_weight, src_offset, n_chunks):
    E = edge_weight.shape[0]
    ep = _NTILES * n_chunks * _C
    pad = ep - E
    src = jnp.concatenate(
        [edge_index[1] + src_offset, jnp.zeros((pad,), jnp.int32)])
    dst = jnp.concatenate([edge_index[0], jnp.zeros((pad,), jnp.int32)])
    w = jnp.concatenate([edge_weight, jnp.zeros((pad,), jnp.float32)])
    return (src.reshape(_NTILES, n_chunks, _C),
            dst.reshape(_NTILES, n_chunks, _C),
            w.reshape(_NTILES, n_chunks, _C))


def kernel(x, edge_index1, edge_weight1, edge_index2, edge_weight2,
           W_in1, b_in1, W_in2, b_in2, fW1, fW2,
           W_out1, b_out1, W_out2, b_out2):
    N, D = x.shape
    E = edge_weight1.shape[0]
    K = fW1.shape[0] - 1
    assert N == _N and W_in1.shape[1] == _H

    n_chunks = -(-E // (_NTILES * _C))
    n_chunks = -(-n_chunks // 3) * 3

    h2n = _input_linear(x, W_in1, b_in1, W_in2, b_in2)
    h_stack = jnp.concatenate([
        jnp.pad(h2n[:N], ((0, _NP - N), (0, 0))),
        jnp.pad(h2n[N:], ((0, _NP - N), (0, 0))),
    ])

    s1, d1, w1 = _prep_edges(edge_index1, edge_weight1, 0, n_chunks)
    s2, d2, w2 = _prep_edges(edge_index2, edge_weight2, _NP, n_chunks)
    src_e = jnp.stack([s1, s2])
    dst_e = jnp.stack([d1, d2])
    w_e = jnp.stack([w1, w2])
    fw = jnp.stack([
        jnp.zeros((16,), jnp.float32).at[:K + 1].set(fW1),
        jnp.zeros((16,), jnp.float32).at[:K + 1].set(fW2),
    ])
    fw = jnp.broadcast_to(fw[:, :, None], (2, 16, 16))

    prop = _make_prop_kernel(K, n_chunks)
    hid = prop(h_stack, src_e, dst_e, w_e, fw)
    hid = jnp.concatenate([hid[:N], hid[_NP:_NP + N]])

    return _output_linear(hid, W_out1, b_out1, W_out2, b_out2)


# speedup vs baseline: 8.5107x; 2.5374x over previous
import functools

import jax
import jax.numpy as jnp
from jax import lax
from jax.experimental import pallas as pl
from jax.experimental.pallas import tpu as pltpu
from jax.experimental.pallas import tpu_sc as plsc

_N = 10000
_NP = 10240
_H = 64
_NTILES = 16
_ROWS_PER_TILE = _NP // _NTILES
_RCH = 128
_C = 128
_NB = 4
_SCH = 16
_NCH = 160
_NSUP = _NCH // _SCH


def _input_linear(x, W1, b1, W2, b2):
    N, D = x.shape
    H = W1.shape[1]
    BN = 1000
    nb = N // BN
    Ws = jnp.stack([W1, W2])
    bs = jnp.stack([b1.reshape(1, H), b2.reshape(1, H)])

    def body(x_ref, w_ref, b_ref, o_ref):
        acc = jnp.dot(x_ref[...], w_ref[0],
                      preferred_element_type=jnp.float32)
        o_ref[...] = acc + b_ref[0]

    return pl.pallas_call(
        body,
        grid=(2, nb),
        in_specs=[
            pl.BlockSpec((BN, D), lambda i, b: (b, 0)),
            pl.BlockSpec((1, D, H), lambda i, b: (i, 0, 0)),
            pl.BlockSpec((1, 1, H), lambda i, b: (i, 0, 0)),
        ],
        out_specs=pl.BlockSpec((BN, H), lambda i, b: (i * nb + b, 0)),
        out_shape=jax.ShapeDtypeStruct((2 * N, H), jnp.float32),
    )(x, Ws, bs)


def _output_linear(hid, Wo1, bo1, Wo2, bo2):
    twoN, H = hid.shape
    N = twoN // 2
    OUT = Wo2.shape[1]
    BN = 1000
    nb = N // BN
    Wa = Wo1[:H]
    Wb = Wo1[H:]

    def body(h1_ref, h2_ref, wa_ref, wb_ref, b1_ref, w2_ref, b2_ref, o_ref):
        t = (jnp.dot(h1_ref[...], wa_ref[...], preferred_element_type=jnp.float32)
             + jnp.dot(h2_ref[...], wb_ref[...], preferred_element_type=jnp.float32)
             + b1_ref[...])
        t = jnp.where(t >= 0, t, 0.01 * t)
        o = jnp.dot(t, w2_ref[...], preferred_element_type=jnp.float32) + b2_ref[...]
        o_ref[...] = jnp.where(o >= 0, o, 0.01 * o)

    return pl.pallas_call(
        body,
        grid=(nb,),
        in_specs=[
            pl.BlockSpec((BN, H), lambda b: (b, 0)),
            pl.BlockSpec((BN, H), lambda b: (nb + b, 0)),
            pl.BlockSpec((H, H), lambda b: (0, 0)),
            pl.BlockSpec((H, H), lambda b: (0, 0)),
            pl.BlockSpec((1, H), lambda b: (0, 0)),
            pl.BlockSpec((H, OUT), lambda b: (0, 0)),
            pl.BlockSpec((1, OUT), lambda b: (0, 0)),
        ],
        out_specs=pl.BlockSpec((BN, OUT), lambda b: (b, 0)),
        out_shape=jax.ShapeDtypeStruct((N, OUT), jnp.float32),
    )(hid, hid, Wa, Wb, bo1.reshape(1, H), Wo2, bo2.reshape(1, OUT))


def _lane_splat(vec, lane):
    idx = jnp.full((16, 1), lane, jnp.int32)
    return lax.gather(
        vec, idx,
        lax.GatherDimensionNumbers(
            offset_dims=(), collapsed_slice_dims=(0,), start_index_map=(0,)),
        (1,), mode=lax.GatherScatterMode.PROMISE_IN_BOUNDS)


def _make_prop_kernel(K):
    mesh = plsc.VectorSubcoreMesh(core_axis_name="c", subcore_axis_name="s")

    @functools.partial(
        pl.kernel,
        out_type=jax.ShapeDtypeStruct((2 * _NP, _H), jnp.float32),
        mesh=mesh,
        scratch_types=[
            pltpu.VMEM_SHARED((2 * _NP, _H), jnp.float32),
            pltpu.VMEM((2, _SCH, _C), jnp.int32),
            pltpu.VMEM((2, _SCH, _C), jnp.int32),
            pltpu.VMEM((2, _SCH, _C), jnp.float32),
            pltpu.VMEM((_NB, _C, _H), jnp.float32),
            pltpu.VMEM((16, 16), jnp.float32),
            pltpu.SemaphoreType.DMA,
            pltpu.SemaphoreType.DMA,
            pltpu.SemaphoreType.DMA,
            pltpu.SemaphoreType.DMA,
            pltpu.SemaphoreType.DMA,
            pltpu.SemaphoreType.DMA,
            pltpu.SemaphoreType.DMA,
            pltpu.SemaphoreType.DMA,
            pltpu.SemaphoreType.DMA,
            pltpu.SemaphoreType.DMA,
        ],
        compiler_params=pltpu.CompilerParams(use_tc_tiling_on_sc=False),
    )
    def prop(h_in, src_e, dst_e, w_e, fw, hid_out,
             h_sh, src_b, dst_b, w_b, rows_v, fw_v,
             gs0, gs1, gs2, gs3, ss0, ss1, ss2, ss3, es0, es1):
        gsems = [gs0, gs1, gs2, gs3]
        ssems = [ss0, ss1, ss2, ss3]
        esems = [es0, es1]
        cid = lax.axis_index("c")
        sid = lax.axis_index("s")
        base = sid * _ROWS_PER_TILE
        gbase = cid * _NP + base

        pltpu.sync_copy(fw.at[cid], fw_v)

        def _zfill2():
            @pl.loop(0, _C)
            def _z(i):
                for f in range(_H // 16):
                    rows_v[2, i, pl.ds(16 * f, 16)] = jnp.zeros((16,), jnp.float32)

        _zfill2()
        fw0 = fw_v[0]
        for c in range(_ROWS_PER_TILE // _RCH):
            r0 = base + c * _RCH
            g0 = gbase + c * _RCH
            pltpu.sync_copy(h_in.at[pl.ds(g0, _RCH)], rows_v.at[0])
            pltpu.sync_copy(rows_v.at[0], h_sh.at[pl.ds(r0, _RCH)])

            @pl.loop(0, _RCH)
            def _scale0(i):
                for f in range(_H // 16):
                    rows_v[1, i, pl.ds(16 * f, 16)] = rows_v[0, i, pl.ds(16 * f, 16)] * fw0

            pltpu.sync_copy(rows_v.at[1], hid_out.at[pl.ds(g0, _RCH)])
            pltpu.sync_copy(rows_v.at[2], h_sh.at[pl.ds(_NP + r0, _RCH)])
        plsc.subcore_barrier()

        def _edge_descs(par, s, eb):
            sl = pl.ds(s * _SCH, _SCH)
            return [
                (src_e.at[par, cid, sid, sl], src_b.at[eb]),
                (dst_e.at[par, cid, sid, sl], dst_b.at[eb]),
                (w_e.at[cid, sid, sl], w_b.at[eb]),
            ]

        def _fire_edges(par, s, eb):
            for a, d in _edge_descs(par, s, eb):
                pltpu.async_copy(a, d, esems[eb])

        def _wait_edges(par, s, eb):
            for a, d in _edge_descs(par, s, eb):
                pltpu.make_async_copy(a, d, esems[eb]).wait()

        @pl.loop(0, K)
        def _round(k):
            par = lax.rem(k, 2)
            _fire_edges(par, 0, 0)

            @pl.loop(0, _NSUP, step=2)
            def _sup(s0):
                for eb in range(2):
                    s = s0 + eb
                    _wait_edges(par, s, eb)

                    @pl.when(s + 1 < _NSUP)
                    def _prefetch():
                        _fire_edges(par, s + 1, 1 - eb)

                    for b in range(_NB):
                        pltpu.async_copy(
                            h_sh.at[src_b.at[eb, b]], rows_v.at[b], gsems[b])

                    @pl.loop(0, _SCH, step=_NB)
                    def _grp(j0):
                        for b in range(_NB):
                            j = j0 + b
                            pltpu.make_async_copy(
                                h_sh.at[src_b.at[eb, j]], rows_v.at[b],
                                gsems[b]).wait()

                            @pl.loop(0, _C // 16, unroll=2)
                            def _scale(g):
                                wv = w_b[eb, j, pl.ds(g * 16, 16)]
                                for e in range(16):
                                    w = _lane_splat(wv, e)
                                    for f in range(_H // 16):
                                        rows_v[b, g * 16 + e, pl.ds(16 * f, 16)] = (
                                            rows_v[b, g * 16 + e, pl.ds(16 * f, 16)] * w)

                            pltpu.async_copy(
                                rows_v.at[b], h_sh.at[dst_b.at[eb, j]],
                                ssems[b], add=True)
                            nxt = j + _NB

                            @pl.when(nxt < _SCH)
                            def _refill():
                                pltpu.make_async_copy(
                                    rows_v.at[b], h_sh.at[dst_b.at[eb, j]],
                                    ssems[b]).wait()
                                pltpu.async_copy(
                                    h_sh.at[src_b.at[eb, nxt]], rows_v.at[b],
                                    gsems[b])

                    for b in range(_NB):
                        pltpu.make_async_copy(
                            rows_v.at[b], h_sh.at[dst_b.at[eb, 0]],
                            ssems[b]).wait()

            plsc.subcore_barrier()

            _zfill2()
            fwk = fw_v[k + 1]
            newoff = (1 - par) * _NP
            oldoff = par * _NP
            for c in range(_ROWS_PER_TILE // _RCH):
                r0 = base + c * _RCH
                g0 = gbase + c * _RCH
                pltpu.sync_copy(h_sh.at[pl.ds(newoff + r0, _RCH)], rows_v.at[1])
                pltpu.sync_copy(hid_out.at[pl.ds(g0, _RCH)], rows_v.at[0])

                @pl.loop(0, _RCH)
                def _fma(i):
                    for f in range(_H // 16):
                        rows_v[0, i, pl.ds(16 * f, 16)] = (
                            rows_v[0, i, pl.ds(16 * f, 16)]
                            + rows_v[1, i, pl.ds(16 * f, 16)] * fwk)

                pltpu.sync_copy(rows_v.at[0], hid_out.at[pl.ds(g0, _RCH)])
                pltpu.sync_copy(rows_v.at[2], h_sh.at[pl.ds(oldoff + r0, _RCH)])
            plsc.subcore_barrier()

    return prop


def _prep_edges(edge_index, edge_weight):
    E = edge_weight.shape[0]
    ep = _NTILES * _NCH * _C
    pad = ep - E
    src = jnp.concatenate([edge_index[1], jnp.zeros((pad,), jnp.int32)])
    dst = jnp.concatenate([edge_index[0], jnp.zeros((pad,), jnp.int32)])
    w = jnp.concatenate([edge_weight, jnp.zeros((pad,), jnp.float32)])
    src = src.reshape(_NTILES, _NCH, _C)
    dst = dst.reshape(_NTILES, _NCH, _C)
    w = w.reshape(_NTILES, _NCH, _C)
    return src, src + _NP, dst + _NP, dst, w


def kernel(x, edge_index1, edge_weight1, edge_index2, edge_weight2,
           W_in1, b_in1, W_in2, b_in2, fW1, fW2,
           W_out1, b_out1, W_out2, b_out2):
    N, D = x.shape
    E = edge_weight1.shape[0]
    K = fW1.shape[0] - 1
    assert N == _N and W_in1.shape[1] == _H
    assert E <= _NTILES * _NCH * _C

    h2n = _input_linear(x, W_in1, b_in1, W_in2, b_in2)
    h_stack = jnp.concatenate([
        jnp.pad(h2n[:N], ((0, _NP - N), (0, 0))),
        jnp.pad(h2n[N:], ((0, _NP - N), (0, 0))),
    ])

    s01, s11, d01, d11, w1 = _prep_edges(edge_index1, edge_weight1)
    s02, s12, d02, d12, w2 = _prep_edges(edge_index2, edge_weight2)
    src_e = jnp.stack([jnp.stack([s01, s02]), jnp.stack([s11, s12])])
    dst_e = jnp.stack([jnp.stack([d01, d02]), jnp.stack([d11, d12])])
    w_e = jnp.stack([w1, w2])
    fw = jnp.stack([
        jnp.zeros((16,), jnp.float32).at[:K + 1].set(fW1),
        jnp.zeros((16,), jnp.float32).at[:K + 1].set(fW2),
    ])
    fw = jnp.broadcast_to(fw[:, :, None], (2, 16, 16))

    prop = _make_prop_kernel(K)
    hid = prop(h_stack, src_e, dst_e, w_e, fw)
    hid = jnp.concatenate([hid[:N], hid[_NP:_NP + N]])

    return _output_linear(hid, W_out1, b_out1, W_out2, b_out2)


# speedup vs baseline: 8.6600x; 1.0175x over previous
import functools

import jax
import jax.numpy as jnp
from jax import lax
from jax.experimental import pallas as pl
from jax.experimental.pallas import tpu as pltpu
from jax.experimental.pallas import tpu_sc as plsc

_N = 10000
_NP = 10240
_H = 64
_NTILES = 16
_ROWS_PER_TILE = _NP // _NTILES
_RCH = 128
_C = 128
_NB = 4
_SCH = 16
_NCH = 160
_NSUP = _NCH // _SCH


def _input_linear(x_pad, W1, b1, W2, b2):
    N, D = x_pad.shape
    H = W1.shape[1]
    BN = 1024
    nb = N // BN
    Ws = jnp.stack([W1, W2])
    bs = jnp.stack([b1.reshape(1, H), b2.reshape(1, H)])

    def body(x_ref, w_ref, b_ref, o_ref):
        acc = jnp.dot(x_ref[...], w_ref[0],
                      preferred_element_type=jnp.float32)
        o_ref[...] = acc + b_ref[0]

    return pl.pallas_call(
        body,
        grid=(2, nb),
        in_specs=[
            pl.BlockSpec((BN, D), lambda i, b: (b, 0)),
            pl.BlockSpec((1, D, H), lambda i, b: (i, 0, 0)),
            pl.BlockSpec((1, 1, H), lambda i, b: (i, 0, 0)),
        ],
        out_specs=pl.BlockSpec((BN, H), lambda i, b: (i * nb + b, 0)),
        out_shape=jax.ShapeDtypeStruct((2 * N, H), jnp.float32),
    )(x_pad, Ws, bs)


def _output_linear(hid, Wo1, bo1, Wo2, bo2):
    H = hid.shape[1]
    OUT = Wo2.shape[1]
    BN = 1024
    nb = _NP // BN
    Wa = Wo1[:H]
    Wb = Wo1[H:]

    def body(h1_ref, h2_ref, wa_ref, wb_ref, b1_ref, w2_ref, b2_ref, o_ref):
        t = (jnp.dot(h1_ref[...], wa_ref[...], preferred_element_type=jnp.float32)
             + jnp.dot(h2_ref[...], wb_ref[...], preferred_element_type=jnp.float32)
             + b1_ref[...])
        t = jnp.where(t >= 0, t, 0.01 * t)
        o = jnp.dot(t, w2_ref[...], preferred_element_type=jnp.float32) + b2_ref[...]
        o_ref[...] = jnp.where(o >= 0, o, 0.01 * o)

    return pl.pallas_call(
        body,
        grid=(nb,),
        in_specs=[
            pl.BlockSpec((BN, H), lambda b: (b, 0)),
            pl.BlockSpec((BN, H), lambda b: (nb + b, 0)),
            pl.BlockSpec((H, H), lambda b: (0, 0)),
            pl.BlockSpec((H, H), lambda b: (0, 0)),
            pl.BlockSpec((1, H), lambda b: (0, 0)),
            pl.BlockSpec((H, OUT), lambda b: (0, 0)),
            pl.BlockSpec((1, OUT), lambda b: (0, 0)),
        ],
        out_specs=pl.BlockSpec((BN, OUT), lambda b: (b, 0)),
        out_shape=jax.ShapeDtypeStruct((_NP, OUT), jnp.float32),
    )(hid, hid, Wa, Wb, bo1.reshape(1, H), Wo2, bo2.reshape(1, OUT))[:_N]


def _lane_splat(vec, lane):
    idx = jnp.full((16, 1), lane, jnp.int32)
    return lax.gather(
        vec, idx,
        lax.GatherDimensionNumbers(
            offset_dims=(), collapsed_slice_dims=(0,), start_index_map=(0,)),
        (1,), mode=lax.GatherScatterMode.PROMISE_IN_BOUNDS)


def _make_prop_kernel(K):
    mesh = plsc.VectorSubcoreMesh(core_axis_name="c", subcore_axis_name="s")

    @functools.partial(
        pl.kernel,
        out_type=jax.ShapeDtypeStruct((2 * _NP, _H), jnp.float32),
        mesh=mesh,
        scratch_types=[
            pltpu.VMEM_SHARED((2 * _NP, _H), jnp.float32),
            pltpu.VMEM((2, _SCH, _C), jnp.int32),
            pltpu.VMEM((2, _SCH, _C), jnp.int32),
            pltpu.VMEM((2, _SCH, _C), jnp.float32),
            pltpu.VMEM((_NB, _C, _H), jnp.float32),
            pltpu.VMEM((16, 16), jnp.float32),
            pltpu.SemaphoreType.DMA,
            pltpu.SemaphoreType.DMA,
            pltpu.SemaphoreType.DMA,
            pltpu.SemaphoreType.DMA,
            pltpu.SemaphoreType.DMA,
            pltpu.SemaphoreType.DMA,
            pltpu.SemaphoreType.DMA,
            pltpu.SemaphoreType.DMA,
            pltpu.SemaphoreType.DMA,
            pltpu.SemaphoreType.DMA,
        ],
        compiler_params=pltpu.CompilerParams(use_tc_tiling_on_sc=False),
    )
    def prop(h_in, src_e, dst_e, w_e, fw, hid_out,
             h_sh, src_b, dst_b, w_b, rows_v, fw_v,
             gs0, gs1, gs2, gs3, ss0, ss1, ss2, ss3, es0, es1):
        gsems = [gs0, gs1, gs2, gs3]
        ssems = [ss0, ss1, ss2, ss3]
        esems = [es0, es1]
        cid = lax.axis_index("c")
        sid = lax.axis_index("s")
        base = sid * _ROWS_PER_TILE
        gbase = cid * _NP + base

        pltpu.sync_copy(fw.at[cid], fw_v)

        def _zfill2():
            @pl.loop(0, _C)
            def _z(i):
                for f in range(_H // 16):
                    rows_v[2, i, pl.ds(16 * f, 16)] = jnp.zeros((16,), jnp.float32)

        _zfill2()
        fw0 = fw_v[0]
        for c in range(_ROWS_PER_TILE // _RCH):
            r0 = base + c * _RCH
            g0 = gbase + c * _RCH
            pltpu.sync_copy(h_in.at[pl.ds(g0, _RCH)], rows_v.at[0])
            pltpu.sync_copy(rows_v.at[0], h_sh.at[pl.ds(r0, _RCH)])

            @pl.loop(0, _RCH)
            def _scale0(i):
                for f in range(_H // 16):
                    rows_v[1, i, pl.ds(16 * f, 16)] = rows_v[0, i, pl.ds(16 * f, 16)] * fw0

            pltpu.sync_copy(rows_v.at[1], hid_out.at[pl.ds(g0, _RCH)])
            pltpu.sync_copy(rows_v.at[2], h_sh.at[pl.ds(_NP + r0, _RCH)])
        plsc.subcore_barrier()

        def _edge_descs(par, s, eb):
            sl = pl.ds(s * _SCH, _SCH)
            return [
                (src_e.at[par, cid, sid, sl], src_b.at[eb]),
                (dst_e.at[par, cid, sid, sl], dst_b.at[eb]),
                (w_e.at[cid, sid, sl], w_b.at[eb]),
            ]

        def _fire_edges(par, s, eb):
            for a, d in _edge_descs(par, s, eb):
                pltpu.async_copy(a, d, esems[eb])

        def _wait_edges(par, s, eb):
            for a, d in _edge_descs(par, s, eb):
                pltpu.make_async_copy(a, d, esems[eb]).wait()

        @pl.loop(0, K)
        def _round(k):
            par = lax.rem(k, 2)
            _fire_edges(par, 0, 0)

            @pl.loop(0, _NSUP, step=2)
            def _sup(s0):
                for eb in range(2):
                    s = s0 + eb
                    _wait_edges(par, s, eb)

                    @pl.when(s + 1 < _NSUP)
                    def _prefetch():
                        _fire_edges(par, s + 1, 1 - eb)

                    for b in range(_NB):
                        pltpu.async_copy(
                            h_sh.at[src_b.at[eb, b]], rows_v.at[b], gsems[b])

                    @pl.loop(0, _SCH, step=_NB)
                    def _grp(j0):
                        for b in range(_NB):
                            j = j0 + b
                            pltpu.make_async_copy(
                                h_sh.at[src_b.at[eb, j]], rows_v.at[b],
                                gsems[b]).wait()

                            @pl.loop(0, _C // 16, unroll=2)
                            def _scale(g):
                                wv = w_b[eb, j, pl.ds(g * 16, 16)]
                                for e in range(16):
                                    w = _lane_splat(wv, e)
                                    for f in range(_H // 16):
                                        rows_v[b, g * 16 + e, pl.ds(16 * f, 16)] = (
                                            rows_v[b, g * 16 + e, pl.ds(16 * f, 16)] * w)

                            pltpu.async_copy(
                                rows_v.at[b], h_sh.at[dst_b.at[eb, j]],
                                ssems[b], add=True)
                            nxt = j + _NB

                            @pl.when(nxt < _SCH)
                            def _refill():
                                pltpu.make_async_copy(
                                    rows_v.at[b], h_sh.at[dst_b.at[eb, j]],
                                    ssems[b]).wait()
                                pltpu.async_copy(
                                    h_sh.at[src_b.at[eb, nxt]], rows_v.at[b],
                                    gsems[b])

                    for b in range(_NB):
                        pltpu.make_async_copy(
                            rows_v.at[b], h_sh.at[dst_b.at[eb, 0]],
                            ssems[b]).wait()

            plsc.subcore_barrier()

            _zfill2()
            fwk = fw_v[k + 1]
            newoff = (1 - par) * _NP
            oldoff = par * _NP
            for c in range(_ROWS_PER_TILE // _RCH):
                r0 = base + c * _RCH
                g0 = gbase + c * _RCH
                pltpu.sync_copy(h_sh.at[pl.ds(newoff + r0, _RCH)], rows_v.at[1])
                pltpu.sync_copy(hid_out.at[pl.ds(g0, _RCH)], rows_v.at[0])

                @pl.loop(0, _RCH)
                def _fma(i):
                    for f in range(_H // 16):
                        rows_v[0, i, pl.ds(16 * f, 16)] = (
                            rows_v[0, i, pl.ds(16 * f, 16)]
                            + rows_v[1, i, pl.ds(16 * f, 16)] * fwk)

                pltpu.sync_copy(rows_v.at[0], hid_out.at[pl.ds(g0, _RCH)])
                pltpu.sync_copy(rows_v.at[2], h_sh.at[pl.ds(oldoff + r0, _RCH)])
            plsc.subcore_barrier()

    return prop


def _prep_edges(edge_index, edge_weight):
    E = edge_weight.shape[0]
    ep = _NTILES * _NCH * _C
    pad = ep - E
    src = jnp.concatenate([edge_index[1], jnp.zeros((pad,), jnp.int32)])
    dst = jnp.concatenate([edge_index[0], jnp.zeros((pad,), jnp.int32)])
    w = jnp.concatenate([edge_weight, jnp.zeros((pad,), jnp.float32)])
    src = src.reshape(_NTILES, _NCH, _C)
    dst = dst.reshape(_NTILES, _NCH, _C)
    w = w.reshape(_NTILES, _NCH, _C)
    return src, src + _NP, dst + _NP, dst, w


def kernel(x, edge_index1, edge_weight1, edge_index2, edge_weight2,
           W_in1, b_in1, W_in2, b_in2, fW1, fW2,
           W_out1, b_out1, W_out2, b_out2):
    N, D = x.shape
    E = edge_weight1.shape[0]
    K = fW1.shape[0] - 1
    assert N == _N and W_in1.shape[1] == _H
    assert E <= _NTILES * _NCH * _C

    x_pad = jnp.pad(x, ((0, _NP - N), (0, 0)))
    h_stack = _input_linear(x_pad, W_in1, b_in1, W_in2, b_in2)

    s01, s11, d01, d11, w1 = _prep_edges(edge_index1, edge_weight1)
    s02, s12, d02, d12, w2 = _prep_edges(edge_index2, edge_weight2)
    src_e = jnp.stack([jnp.stack([s01, s02]), jnp.stack([s11, s12])])
    dst_e = jnp.stack([jnp.stack([d01, d02]), jnp.stack([d11, d12])])
    w_e = jnp.stack([w1, w2])
    fw = jnp.stack([
        jnp.zeros((16,), jnp.float32).at[:K + 1].set(fW1),
        jnp.zeros((16,), jnp.float32).at[:K + 1].set(fW2),
    ])
    fw = jnp.broadcast_to(fw[:, :, None], (2, 16, 16))

    prop = _make_prop_kernel(K)
    hid = prop(h_stack, src_e, dst_e, w_e, fw)

    return _output_linear(hid, W_out1, b_out1, W_out2, b_out2)
